# baseline (speedup 1.0000x reference)
import functools

import jax
import jax.numpy as jnp
from jax import lax
from jax.experimental import pallas as pl
from jax.experimental.pallas import tpu as pltpu
from jax.experimental.pallas import tpu_sc as plsc

_NC = 2
_NS = 16
_NW = _NC * _NS
_C = 32


def _sc_body(tok_hbm, pos_hbm, out_hbm, pos_buf, tok_buf, B, T, D):
    span = T // _NW
    n_chunks = span // _C
    chunk_elems = _C * D
    wid = lax.axis_index("s") * _NC + lax.axis_index("c")
    t0 = wid * span

    def t_chunk(tc, carry):
        pos_off = (t0 + tc * _C) * D
        pltpu.sync_copy(pos_hbm.at[pl.ds(pos_off, chunk_elems)], pos_buf)

        def batch(b, c):
            off = b * (T * D) + pos_off
            pltpu.sync_copy(tok_hbm.at[pl.ds(off, chunk_elems)], tok_buf)

            def add16(i, _=None):
                sl = pl.ds(i * 16, 16)
                plsc.addupdate(tok_buf.at[sl], pos_buf[sl])

            plsc.parallel_loop(0, chunk_elems // 16, 1, unroll=8)(add16)
            pltpu.sync_copy(tok_buf, out_hbm.at[pl.ds(off, chunk_elems)])
            return c

        lax.fori_loop(0, B, batch, 0)
        return carry

    lax.fori_loop(0, n_chunks, t_chunk, 0)


def kernel(tokens, pos_table):
    B, T, D = tokens.shape
    tok_flat = tokens.reshape(B * T * D)
    pos_flat = pos_table.reshape(T * D)

    sc_add = pl.kernel(
        functools.partial(_sc_body, B=B, T=T, D=D),
        out_type=jax.ShapeDtypeStruct((B * T * D,), jnp.float32),
        mesh=plsc.VectorSubcoreMesh(core_axis_name="c", subcore_axis_name="s"),
        scratch_types=[
            pltpu.VMEM((_C * D,), jnp.float32),
            pltpu.VMEM((_C * D,), jnp.float32),
        ],
    )

    out = sc_add(tok_flat, pos_flat)
    return out.reshape(B, T, D)


# speedup vs baseline: 1.2096x; 1.2096x over previous
import functools

import jax
import jax.numpy as jnp
from jax import lax
from jax.experimental import pallas as pl
from jax.experimental.pallas import tpu as pltpu
from jax.experimental.pallas import tpu_sc as plsc

_NC = 2
_NS = 16
_NW = _NC * _NS
_C = 16
_K = 4
_A = 2


def _sc_body(tok_hbm, pos_hbm, out_hbm,
             t0_buf, t1_buf, t2_buf, t3_buf, p0_buf, p1_buf,
             si0, si1, si2, si3, so0, so1, so2, so3, sp0, sp1,
             B, T, D):
    span = T // _NW
    n_chunks = span // _C
    ce = _C * D
    n_steps = n_chunks * B
    wid = lax.axis_index("s") * _NC + lax.axis_index("c")
    t0 = wid * span

    tok = [t0_buf, t1_buf, t2_buf, t3_buf]
    pos = [p0_buf, p1_buf]
    sin = [si0, si1, si2, si3]
    sout = [so0, so1, so2, so3]
    spos = [sp0, sp1]

    din, dout, dpos = {}, {}, {}

    def start_in(s):
        tc, b = divmod(s, B)
        off = b * (T * D) + (t0 + tc * _C) * D
        din[s] = pltpu.async_copy(
            tok_hbm.at[pl.ds(off, ce)], tok[s % _K], sin[s % _K])

    def start_out(s):
        tc, b = divmod(s, B)
        off = b * (T * D) + (t0 + tc * _C) * D
        dout[s] = pltpu.async_copy(
            tok[s % _K], out_hbm.at[pl.ds(off, ce)], sout[s % _K])

    def start_pos(tc):
        dpos[tc] = pltpu.async_copy(
            pos_hbm.at[pl.ds((t0 + tc * _C) * D, ce)], pos[tc % 2], spos[tc % 2])

    start_pos(0)
    for s in range(_A):
        start_in(s)

    for s in range(n_steps):
        tc, b = divmod(s, B)
        if b == 0:
            dpos[tc].wait()
            if tc + 1 < n_chunks:
                start_pos(tc + 1)
        din[s].wait()

        tbuf, pbuf = tok[s % _K], pos[tc % 2]

        def add16(i, _=None, tbuf=tbuf, pbuf=pbuf):
            sl = pl.ds(i * 16, 16)
            plsc.addupdate(tbuf.at[sl], pbuf[sl])

        plsc.parallel_loop(0, ce // 16, 1, unroll=8)(add16)
        start_out(s)
        if s + _A < n_steps:
            if s - _A >= 0:
                dout[s - _A].wait()
            start_in(s + _A)

    for s in range(n_steps - 2 * _A, n_steps):
        if s >= 0:
            dout[s].wait()


def kernel(tokens, pos_table):
    B, T, D = tokens.shape
    tok_flat = tokens.reshape(B * T * D)
    pos_flat = pos_table.reshape(T * D)

    sc_add = pl.kernel(
        functools.partial(_sc_body, B=B, T=T, D=D),
        out_type=jax.ShapeDtypeStruct((B * T * D,), jnp.float32),
        mesh=plsc.VectorSubcoreMesh(core_axis_name="c", subcore_axis_name="s"),
        scratch_types=(
            [pltpu.VMEM((_C * D,), jnp.float32)] * (_K + 2)
            + [pltpu.SemaphoreType.DMA] * (2 * _K + 2)
        ),
    )

    out = sc_add(tok_flat, pos_flat)
    return out.reshape(B, T, D)


# speedup vs baseline: 3.3792x; 2.7937x over previous
import functools

import jax
import jax.numpy as jnp
from jax import lax
from jax.experimental import pallas as pl
from jax.experimental.pallas import tpu as pltpu
from jax.experimental.pallas import tpu_sc as plsc

_NC = 2
_NS = 16
_NW = _NC * _NS
_C = 16
_K = 4
_A = 2


def _sc_body(tok_hbm, pos_hbm, out_hbm,
             t0_buf, t1_buf, t2_buf, t3_buf, p0_buf, p1_buf,
             si0, si1, si2, si3, so0, so1, so2, so3, sp0, sp1,
             B, T, D):
    span = T // _NW
    n_chunks = span // _C
    n_steps = n_chunks * B
    wid = lax.axis_index("s") * _NC + lax.axis_index("c")
    t0 = wid * span

    tok = [t0_buf, t1_buf, t2_buf, t3_buf]
    pos = [p0_buf, p1_buf]
    sin = [si0, si1, si2, si3]
    sout = [so0, so1, so2, so3]
    spos = [sp0, sp1]

    din, dout, dpos = {}, {}, {}

    def start_in(s):
        tc, b = divmod(s, B)
        tt = t0 + tc * _C
        din[s] = pltpu.async_copy(
            tok_hbm.at[b, pl.ds(tt, _C)], tok[s % _K], sin[s % _K])

    def start_out(s):
        tc, b = divmod(s, B)
        tt = t0 + tc * _C
        dout[s] = pltpu.async_copy(
            tok[s % _K], out_hbm.at[b, pl.ds(tt, _C)], sout[s % _K])

    def start_pos(tc):
        dpos[tc] = pltpu.async_copy(
            pos_hbm.at[pl.ds(t0 + tc * _C, _C)], pos[tc % 2], spos[tc % 2])

    start_pos(0)
    for s in range(_A):
        start_in(s)

    for s in range(n_steps):
        tc, b = divmod(s, B)
        if b == 0:
            dpos[tc].wait()
            if tc + 1 < n_chunks:
                start_pos(tc + 1)
        din[s].wait()

        tbuf, pbuf = tok[s % _K], pos[tc % 2]

        def row(r, _, tbuf=tbuf, pbuf=pbuf):
            def add16(i, _=None):
                sl = pl.ds(i * 16, 16)
                plsc.addupdate(tbuf.at[r, sl], pbuf[r, sl])

            plsc.parallel_loop(0, D // 16, 1, unroll=8)(add16)
            return _

        lax.fori_loop(0, _C, row, 0)
        start_out(s)
        if s + _A < n_steps:
            if s - _A >= 0:
                dout[s - _A].wait()
            start_in(s + _A)

    for s in range(n_steps - 2 * _A, n_steps):
        if s >= 0:
            dout[s].wait()


def kernel(tokens, pos_table):
    B, T, D = tokens.shape

    sc_add = pl.kernel(
        functools.partial(_sc_body, B=B, T=T, D=D),
        out_type=jax.ShapeDtypeStruct((B, T, D), jnp.float32),
        mesh=plsc.VectorSubcoreMesh(core_axis_name="c", subcore_axis_name="s"),
        scratch_types=(
            [pltpu.VMEM((_C, D), jnp.float32)] * (_K + 2)
            + [pltpu.SemaphoreType.DMA] * (2 * _K + 2)
        ),
    )

    return sc_add(tokens, pos_table)


# speedup vs baseline: 3.6719x; 1.0866x over previous
import functools

import jax
import jax.numpy as jnp
from jax import lax
from jax.experimental import pallas as pl
from jax.experimental.pallas import tpu as pltpu
from jax.experimental.pallas import tpu_sc as plsc

_NC = 2
_NS = 16
_NW = _NC * _NS
_C = 32
_K = 3
_A = 2


def _sc_body(tok_hbm, pos_hbm, out_hbm, *scratch, B, T, D):
    bufs, sems = scratch[:_K + 2], scratch[_K + 2:]
    tok, pos = list(bufs[:_K]), list(bufs[_K:])
    sin, sout, spos = list(sems[:_K]), list(sems[_K:2 * _K]), list(sems[2 * _K:])
    span = T // _NW
    n_chunks = span // _C
    n_steps = n_chunks * B
    wid = lax.axis_index("s") * _NC + lax.axis_index("c")
    t0 = wid * span

    din, dout, dpos = {}, {}, {}

    def start_in(s):
        tc, b = divmod(s, B)
        tt = t0 + tc * _C
        din[s] = pltpu.async_copy(
            tok_hbm.at[b, pl.ds(tt, _C)], tok[s % _K], sin[s % _K])

    def start_out(s):
        tc, b = divmod(s, B)
        tt = t0 + tc * _C
        dout[s] = pltpu.async_copy(
            tok[s % _K], out_hbm.at[b, pl.ds(tt, _C)], sout[s % _K])

    def start_pos(tc):
        dpos[tc] = pltpu.async_copy(
            pos_hbm.at[pl.ds(t0 + tc * _C, _C)], pos[tc % 2], spos[tc % 2])

    start_pos(0)
    for s in range(_A):
        start_in(s)

    for s in range(n_steps):
        tc, b = divmod(s, B)
        if b == 0:
            dpos[tc].wait()
            if tc + 1 < n_chunks:
                start_pos(tc + 1)
        din[s].wait()

        tbuf, pbuf = tok[s % _K], pos[tc % 2]

        def row(r, _, tbuf=tbuf, pbuf=pbuf):
            def add16(i, _=None):
                sl = pl.ds(i * 16, 16)
                plsc.addupdate(tbuf.at[r, sl], pbuf[r, sl])

            plsc.parallel_loop(0, D // 16, 1, unroll=8)(add16)
            return _

        lax.fori_loop(0, _C, row, 0)
        start_out(s)
        if s + _A < n_steps:
            if s - _A >= 0:
                dout[s - _A].wait()
            start_in(s + _A)

    for s in range(n_steps - 2 * _A, n_steps):
        if s >= 0:
            dout[s].wait()


def kernel(tokens, pos_table):
    B, T, D = tokens.shape

    sc_add = pl.kernel(
        functools.partial(_sc_body, B=B, T=T, D=D),
        out_type=jax.ShapeDtypeStruct((B, T, D), jnp.float32),
        mesh=plsc.VectorSubcoreMesh(core_axis_name="c", subcore_axis_name="s"),
        scratch_types=(
            [pltpu.VMEM((_C, D), jnp.float32)] * (_K + 2)
            + [pltpu.SemaphoreType.DMA] * (2 * _K + 2)
        ),
    )

    return sc_add(tokens, pos_table)
